# baseline (speedup 1.0000x reference)
import functools

import jax
import jax.numpy as jnp
from jax import lax
from jax.experimental import pallas as pl
from jax.experimental.pallas import tpu as pltpu
from jax.experimental.pallas import tpu_sc as plsc

BN = 2000
SC_NW = 32
SC_CHUNK = 128


def _pass1_body(h, w, thresh_ref, cls_ref, anc_ref, reg_ref, packed_ref,
                carry_ref, le_ref):
    b = pl.program_id(0)
    nb = pl.program_id(1)

    @pl.when(jnp.logical_and(b == 0, nb == 0))
    def _():
        ii = lax.broadcasted_iota(jnp.int32, (BN, BN), 0)
        jj = lax.broadcasted_iota(jnp.int32, (BN, BN), 1)
        le_ref[...] = (ii <= jj).astype(jnp.float32)

    @pl.when(nb == 0)
    def _():
        carry_ref[0] = 0.0

    x = cls_ref[0]
    c = x.shape[1]
    score = jnp.max(x, axis=1, keepdims=True)
    iota_c = lax.broadcasted_iota(jnp.int32, x.shape, 1)
    amax = jnp.min(jnp.where(x == score, iota_c, c), axis=1, keepdims=True)

    a = anc_ref[0]
    r = reg_ref[0]
    aw = a[:, 2:3] - a[:, 0:1]
    ah = a[:, 3:4] - a[:, 1:2]
    cx = a[:, 0:1] + 0.5 * aw
    cy = a[:, 1:2] + 0.5 * ah
    pcx = cx + r[:, 0:1] * 0.1 * aw
    pcy = cy + r[:, 1:2] * 0.1 * ah
    pw = jnp.exp(r[:, 2:3] * 0.2) * aw
    ph = jnp.exp(r[:, 3:4] * 0.2) * ah
    x1 = jnp.clip(pcx - 0.5 * pw, 0.0, w)
    y1 = jnp.clip(pcy - 0.5 * ph, 0.0, h)
    x2 = jnp.clip(pcx + 0.5 * pw, 0.0, w)
    y2 = jnp.clip(pcy + 0.5 * ph, 0.0, h)

    maskf = (score > thresh_ref[0, 0]).astype(jnp.float32)

    m = jnp.concatenate(
        [x1, y1, x2, y2, score, amax.astype(jnp.float32), maskf], axis=1)
    mt = jnp.transpose(m)
    mask_row = mt[6:7]
    psum_row = jnp.dot(mask_row, le_ref[...],
                       preferred_element_type=jnp.float32) + carry_ref[0]
    carry_ref[0] = carry_ref[0] + jnp.sum(maskf)

    packed_ref[0] = jnp.concatenate([mt[0:6], psum_row, mask_row], axis=0)


def _pass2_body(n, nb_per_img, pk_ref, tlast_ref, dest_ref):
    g = pl.program_id(0)
    b = g // nb_per_img
    nb = g - b * nb_per_img
    p = pk_ref[0]
    score_mask = p[7:8] > 0.0
    psum = p[6:7]
    t = tlast_ref[0, 0, 0]
    row = (lax.broadcasted_iota(jnp.int32, (1, BN), 1).astype(jnp.float32)
           + (nb * BN).astype(jnp.float32))
    dest = jnp.where(score_mask, psum - 1.0, t + row - psum)
    gdest = dest + (b * n).astype(jnp.float32)
    dest_ref[0] = gdest.astype(jnp.int32)


def _sc_scatter_body(nblk, bpad, nch, packed_hbm, gdest_hbm, out_hbm, comp_v,
                     rows_v, idx_v, sem):
    wid = lax.axis_index("s") * 2 + lax.axis_index("c")

    def do_block(gi):
        pltpu.sync_copy(packed_hbm.at[gi], comp_v)
        pltpu.sync_copy(gdest_hbm.at[gi], idx_v)
        nvec = BN // 16
        for ci in range(8):
            col = jnp.full((16,), ci, jnp.int32)

            def body(k, _):
                vec = comp_v[ci, pl.ds(k * 16, 16)]
                rows = k * 16 + lax.broadcasted_iota(jnp.int32, (16,), 0)
                plsc.store_scatter(rows_v, [rows, col], vec)
                return 0

            lax.fori_loop(0, nvec, body, 0)
        cps = []
        for j in range(nch):
            cps.append(
                pltpu.async_copy(rows_v.at[pl.ds(j * SC_CHUNK, SC_CHUNK)],
                                 out_hbm.at[idx_v.at[j]], sem))
        for cp in cps:
            cp.wait()

    do_block(wid)

    @pl.when(wid + SC_NW < nblk)
    def _():
        do_block(wid + SC_NW)


def kernel(imgs, classifications, regressions, anchors, cls_thresh):
    batch, _, height, width = imgs.shape
    _, n, c = classifications.shape
    nb_per_img = n // BN
    g = batch * nb_per_img
    total = batch * n

    thresh = jnp.broadcast_to(cls_thresh.astype(jnp.float32), (8, 128))

    packed_t = pl.pallas_call(
        functools.partial(_pass1_body, float(height), float(width)),
        grid=(batch, nb_per_img),
        in_specs=[
            pl.BlockSpec((8, 128), lambda b, nb: (0, 0)),
            pl.BlockSpec((1, BN, c), lambda b, nb: (b, nb, 0)),
            pl.BlockSpec((1, BN, 4), lambda b, nb: (b, nb, 0)),
            pl.BlockSpec((1, BN, 4), lambda b, nb: (b, nb, 0)),
        ],
        out_specs=pl.BlockSpec((1, 8, BN),
                               lambda b, nb: (b * (n // BN) + nb, 0, 0)),
        out_shape=jax.ShapeDtypeStruct((g, 8, BN), jnp.float32),
        scratch_shapes=[pltpu.SMEM((1,), jnp.float32),
                        pltpu.VMEM((BN, BN), jnp.float32)],
    )(thresh, classifications, anchors, regressions)

    tlast = packed_t[nb_per_img - 1::nb_per_img, 6, BN - 1].reshape(
        batch, 1, 1)

    dest = pl.pallas_call(
        functools.partial(_pass2_body, n, nb_per_img),
        grid=(g,),
        in_specs=[
            pl.BlockSpec((1, 8, BN), lambda i: (i, 0, 0)),
            pl.BlockSpec((1, 1, 1), lambda i: (i // nb_per_img, 0, 0)),
        ],
        out_specs=pl.BlockSpec((1, 1, BN), lambda i: (i, 0, 0)),
        out_shape=jax.ShapeDtypeStruct((g, 1, BN), jnp.int32),
    )(packed_t, tlast)

    bpad = -(-BN // SC_CHUNK) * SC_CHUNK
    nch = bpad // SC_CHUNK
    ppb = bpad - BN
    out_rows = total + g * ppb

    pad_dest = (total + ppb * jnp.arange(g, dtype=jnp.int32)[:, None]
                + jnp.arange(ppb, dtype=jnp.int32)[None, :])
    gdest_pad = jnp.concatenate([dest.reshape(g, BN), pad_dest],
                                axis=1).reshape(g, nch, SC_CHUNK)

    sc_fn = functools.partial(
        pl.kernel,
        mesh=plsc.VectorSubcoreMesh(core_axis_name="c", subcore_axis_name="s"),
        out_type=jax.ShapeDtypeStruct((out_rows, 8), jnp.float32),
        scratch_types=[
            pltpu.VMEM((8, BN), jnp.float32),
            pltpu.VMEM((bpad, 8), jnp.float32),
            pltpu.VMEM((nch, SC_CHUNK), jnp.int32),
            pltpu.SemaphoreType.DMA,
        ],
        compiler_params=pltpu.CompilerParams(use_tc_tiling_on_sc=False,
                                             needs_layout_passes=False),
    )(functools.partial(_sc_scatter_body, g, bpad, nch))

    out = sc_fn(packed_t, gdest_pad)
    res = out[:total].reshape(batch, n, 8)

    boxes = tuple(res[i, :, 0:4] for i in range(batch))
    cls = tuple(res[i, :, 5].astype(jnp.int32) for i in range(batch))
    scores = tuple(res[i, :, 4] for i in range(batch))
    return (boxes, cls, scores)


# speedup vs baseline: 1.0534x; 1.0534x over previous
import functools

import jax
import jax.numpy as jnp
from jax import lax
from jax.experimental import pallas as pl
from jax.experimental.pallas import tpu as pltpu
from jax.experimental.pallas import tpu_sc as plsc

BN = 2000
SC_NW = 32
SC_CHUNK = 128


def _pass1_body(h, w, thresh_ref, cls_ref, anc_ref, reg_ref, packed_ref,
                carry_ref, le_ref):
    b = pl.program_id(0)
    nb = pl.program_id(1)

    @pl.when(jnp.logical_and(b == 0, nb == 0))
    def _():
        ii = lax.broadcasted_iota(jnp.int32, (BN, BN), 0)
        jj = lax.broadcasted_iota(jnp.int32, (BN, BN), 1)
        le_ref[...] = (ii <= jj).astype(jnp.float32)

    @pl.when(nb == 0)
    def _():
        carry_ref[0] = 0.0

    x = cls_ref[0]
    c = x.shape[1]
    score = jnp.max(x, axis=1, keepdims=True)
    iota_c = lax.broadcasted_iota(jnp.int32, x.shape, 1)
    amax = jnp.min(jnp.where(x == score, iota_c, c), axis=1, keepdims=True)

    a = anc_ref[0]
    r = reg_ref[0]
    aw = a[:, 2:3] - a[:, 0:1]
    ah = a[:, 3:4] - a[:, 1:2]
    cx = a[:, 0:1] + 0.5 * aw
    cy = a[:, 1:2] + 0.5 * ah
    pcx = cx + r[:, 0:1] * 0.1 * aw
    pcy = cy + r[:, 1:2] * 0.1 * ah
    pw = jnp.exp(r[:, 2:3] * 0.2) * aw
    ph = jnp.exp(r[:, 3:4] * 0.2) * ah
    x1 = jnp.clip(pcx - 0.5 * pw, 0.0, w)
    y1 = jnp.clip(pcy - 0.5 * ph, 0.0, h)
    x2 = jnp.clip(pcx + 0.5 * pw, 0.0, w)
    y2 = jnp.clip(pcy + 0.5 * ph, 0.0, h)

    maskf = (score > thresh_ref[0, 0]).astype(jnp.float32)

    m = jnp.concatenate(
        [x1, y1, x2, y2, score, amax.astype(jnp.float32), maskf], axis=1)
    mt = jnp.transpose(m)
    mask_row = mt[6:7]
    psum_row = jnp.dot(mask_row, le_ref[...],
                       preferred_element_type=jnp.float32) + carry_ref[0]
    carry_ref[0] = carry_ref[0] + jnp.sum(maskf)

    packed_ref[0] = jnp.concatenate([mt[0:6], psum_row, mask_row], axis=0)


def _pass2_body(n, nb_per_img, pk_ref, tlast_ref, dest_ref):
    g = pl.program_id(0)
    b = g // nb_per_img
    nb = g - b * nb_per_img
    p = pk_ref[0]
    score_mask = p[7:8] > 0.0
    psum = p[6:7]
    t = tlast_ref[0, 0, 0]
    row = (lax.broadcasted_iota(jnp.int32, (1, BN), 1).astype(jnp.float32)
           + (nb * BN).astype(jnp.float32))
    dest = jnp.where(score_mask, psum - 1.0, t + row - psum)
    gdest = dest + (b * n).astype(jnp.float32)
    dest_ref[0] = gdest.astype(jnp.int32)


def _sc_scatter_body(nblk, nbpi, n, tot, ppb, bpad, nch, packed_hbm, out_hbm,
                     comp_v, rows_v, idx_v, tv, sem):
    wid = lax.axis_index("s") * 2 + lax.axis_index("c")
    iota16 = lax.broadcasted_iota(jnp.int32, (16,), 0)

    def do_block(gi):
        b = gi // nbpi
        pltpu.sync_copy(packed_hbm.at[gi], comp_v)
        pltpu.sync_copy(
            packed_hbm.at[(b + 1) * nbpi - 1, 6, pl.ds(BN - 16, 16)], tv)
        t = jnp.max(tv[...])
        base_row = (gi - b * nbpi) * BN
        boff = (b * n).astype(jnp.float32)
        nvec = BN // 16

        def dbody(k, _):
            psv = comp_v[6, pl.ds(k * 16, 16)]
            mkv = comp_v[7, pl.ds(k * 16, 16)]
            rowf = (base_row + k * 16 + iota16).astype(jnp.float32)
            dv = jnp.where(mkv > 0.0, psv - 1.0, t + rowf - psv) + boff
            j = k // 8
            idx_v[j, pl.ds((k - j * 8) * 16, 16)] = dv.astype(jnp.int32)
            return 0

        lax.fori_loop(0, nvec, dbody, 0)
        for kp in range(nvec, bpad // 16):
            jp = kp // 8
            idx_v[jp, pl.ds((kp - jp * 8) * 16, 16)] = (
                tot + gi * ppb + (kp * 16 - BN) + iota16)
        for ci in range(8):
            col = jnp.full((16,), ci, jnp.int32)

            def body(k, _):
                vec = comp_v[ci, pl.ds(k * 16, 16)]
                rows = k * 16 + lax.broadcasted_iota(jnp.int32, (16,), 0)
                plsc.store_scatter(rows_v, [rows, col], vec)
                return 0

            lax.fori_loop(0, nvec, body, 0)
        cps = []
        for j in range(nch):
            cps.append(
                pltpu.async_copy(rows_v.at[pl.ds(j * SC_CHUNK, SC_CHUNK)],
                                 out_hbm.at[idx_v.at[j]], sem))
        for cp in cps:
            cp.wait()

    do_block(wid)

    @pl.when(wid + SC_NW < nblk)
    def _():
        do_block(wid + SC_NW)


def kernel(imgs, classifications, regressions, anchors, cls_thresh):
    batch, _, height, width = imgs.shape
    _, n, c = classifications.shape
    nb_per_img = n // BN
    g = batch * nb_per_img
    total = batch * n

    thresh = jnp.broadcast_to(cls_thresh.astype(jnp.float32), (8, 128))

    packed_t = pl.pallas_call(
        functools.partial(_pass1_body, float(height), float(width)),
        grid=(batch, nb_per_img),
        in_specs=[
            pl.BlockSpec((8, 128), lambda b, nb: (0, 0)),
            pl.BlockSpec((1, BN, c), lambda b, nb: (b, nb, 0)),
            pl.BlockSpec((1, BN, 4), lambda b, nb: (b, nb, 0)),
            pl.BlockSpec((1, BN, 4), lambda b, nb: (b, nb, 0)),
        ],
        out_specs=pl.BlockSpec((1, 8, BN),
                               lambda b, nb: (b * (n // BN) + nb, 0, 0)),
        out_shape=jax.ShapeDtypeStruct((g, 8, BN), jnp.float32),
        scratch_shapes=[pltpu.SMEM((1,), jnp.float32),
                        pltpu.VMEM((BN, BN), jnp.float32)],
    )(thresh, classifications, anchors, regressions)

    bpad = -(-BN // SC_CHUNK) * SC_CHUNK
    nch = bpad // SC_CHUNK
    ppb = bpad - BN
    out_rows = total + g * ppb

    sc_fn = functools.partial(
        pl.kernel,
        mesh=plsc.VectorSubcoreMesh(core_axis_name="c", subcore_axis_name="s"),
        out_type=jax.ShapeDtypeStruct((out_rows, 8), jnp.float32),
        scratch_types=[
            pltpu.VMEM((8, BN), jnp.float32),
            pltpu.VMEM((bpad, 8), jnp.float32),
            pltpu.VMEM((nch, SC_CHUNK), jnp.int32),
            pltpu.VMEM((16,), jnp.float32),
            pltpu.SemaphoreType.DMA,
        ],
        compiler_params=pltpu.CompilerParams(use_tc_tiling_on_sc=False,
                                             needs_layout_passes=False),
    )(functools.partial(_sc_scatter_body, g, nb_per_img, n, total, ppb, bpad,
                        nch))

    out = sc_fn(packed_t)
    res = out[:total].reshape(batch, n, 8)

    boxes = tuple(res[i, :, 0:4] for i in range(batch))
    cls = tuple(res[i, :, 5].astype(jnp.int32) for i in range(batch))
    scores = tuple(res[i, :, 4] for i in range(batch))
    return (boxes, cls, scores)
